# baseline (speedup 1.0000x reference)
import numpy as np
import jax
import jax.numpy as jnp
from jax import lax
from jax.experimental import pallas as pl
from jax.experimental.pallas import tpu as pltpu
from jax.experimental.pallas import tpu_sc as plsc

_N = 10000
_E = 320000
_D = 128
_DROP = int(0.7 * _E)

_C = 128
_NW = 32
_NPAD = 10240
_RPT = _NPAD // 16
_ZR = 80


def _threefry2x32(k0, k1, x0, x1):
    rotations = ((13, 15, 26, 6), (17, 29, 16, 24))
    k0 = np.uint32(k0); k1 = np.uint32(k1)
    ks = (k0, k1, k0 ^ k1 ^ np.uint32(0x1BD11BDA))
    x0 = (x0 + ks[0]).astype(np.uint32)
    x1 = (x1 + ks[1]).astype(np.uint32)
    for i in range(5):
        for r in rotations[i % 2]:
            x0 = (x0 + x1).astype(np.uint32)
            x1 = ((x1 << np.uint32(r)) | (x1 >> np.uint32(32 - r))).astype(np.uint32)
            x1 = x0 ^ x1
        x0 = (x0 + ks[(i + 1) % 3]).astype(np.uint32)
        x1 = (x1 + ks[(i + 2) % 3] + np.uint32(i + 1)).astype(np.uint32)
    return x0, x1


def _kept_indices() -> np.ndarray:
    def bits32(k, n):
        o0, o1 = _threefry2x32(k[0], k[1], np.zeros(n, np.uint32),
                               np.arange(n, dtype=np.uint32))
        return o0 ^ o1

    s0, s1 = _threefry2x32(np.uint32(0), np.uint32(1),
                           np.array([0, 0], np.uint32),
                           np.array([0, 1], np.uint32))
    k_hi = (s0[0], s1[0])
    k_lo = (s0[1], s1[1])
    span = np.uint32(_E)
    hi = bits32(k_hi, _DROP)
    lo = bits32(k_lo, _DROP)
    mult64 = (np.uint64(65536) % np.uint64(span)) ** 2 % np.uint64(2 ** 32)
    mult = np.uint32(mult64 % np.uint64(span))
    ridx = ((hi % span) * mult + (lo % span)).astype(np.uint32) % span
    mask = np.zeros((_E,), dtype=bool)
    mask[ridx] = True
    return np.nonzero(~mask)[0].astype(np.int32)


_KEPT = _kept_indices()
_K = int(_KEPT.shape[0])
_KP = ((_K + _NW * _C - 1) // (_NW * _C)) * (_NW * _C)
_PAD = _KP - _K
_PAD_NODE = ((np.arange(_PAD, dtype=np.int64) * 37) % _N).astype(np.int32)
_KEPT_PADDED = np.concatenate([_KEPT, _E + np.arange(_PAD, dtype=np.int32)])
_NCHUNKS = _KP // (_NW * _C)


def _zero_acc(zbuf, acc, row0):
    def zrow(i, carry):
        z = jnp.zeros((16,), jnp.float32)
        for j in range(8):
            zbuf[i, pl.ds(j * 16, 16)] = z
        return carry

    lax.fori_loop(0, _ZR, zrow, 0)
    for k in range(_RPT // _ZR):
        pltpu.sync_copy(zbuf, acc.at[pl.ds(row0 + k * _ZR, _ZR)])


def _scale_rows(rows, cv):
    @plsc.parallel_loop(0, _C // 16, 1)
    def scale16(g):
        cvec = cv[pl.ds(g * 16, 16)]
        for l in range(16):
            b = jnp.full((16,), cvec[l], jnp.float32)
            e = g * 16 + l
            for j in range(8):
                sl = pl.ds(j * 16, 16)
                rows[e, sl] = rows[e, sl] * b


def _rsqrt16(x):
    i = plsc.bitcast(x, jnp.int32)
    i = 0x5F3759DF - jnp.right_shift(i, 1)
    y = plsc.bitcast(i, jnp.float32)
    for _ in range(3):
        y = y * (1.5 - 0.5 * x * y * y)
    return y


_EPT = _E // 16
_EB = 2000
_RN = _NPAD // 16


def _filter_body(kept_hbm, src_hbm, dst_hbm, w_hbm,
                 fpk_hbm, fc1_hbm, fc2_hbm, innorm_hbm,
                 keptv0, keptv1, srcv0, srcv1, dstv0, dstv1, wv0, wv1,
                 cv, c2v, pkv, ntab1, ntab2, hsrc, hdst, ebuf, ebuf2,
                 ds1, ds2, tmp, hall, sn1, sn2, sem):
    cid = lax.axis_index("c")
    sid = lax.axis_index("s")
    wid = sid * 2 + cid

    keptv = (keptv0, keptv1)
    srcv = (srcv0, srcv1)
    dstv = (dstv0, dstv1)
    wv = (wv0, wv1)

    def zrow(i, carry):
        z = jnp.zeros((16,), jnp.float32)
        hsrc[pl.ds(i * 16, 16)] = z
        hdst[pl.ds(i * 16, 16)] = z
        return carry

    lax.fori_loop(0, _NPAD // 16, zrow, 0)

    ones = jnp.ones((16,), jnp.float32)

    def hist16(buf, hist):
        @pl.loop(0, _EB // 16, unroll=4)
        def h16(g):
            idx = buf[pl.ds(g * 16, 16)]
            plsc.addupdate_scatter(hist, [idx], ones)

    pltpu.sync_copy(src_hbm.at[pl.ds(sid * _EPT, _EB)], ebuf)

    @pl.loop(0, _EPT // _EB)
    def hist_chunk(c):
        base = sid * _EPT + c * _EB
        d = pltpu.async_copy(dst_hbm.at[pl.ds(base, _EB)], ebuf2, sem)
        hist16(ebuf, hsrc)
        d.wait()

        @pl.when(c + 1 < _EPT // _EB)
        def _():
            d2 = pltpu.async_copy(
                src_hbm.at[pl.ds(base + _EB, _EB)], ebuf, sem)
            hist16(ebuf2, hdst)
            d2.wait()

        @pl.when(c + 1 >= _EPT // _EB)
        def _():
            hist16(ebuf2, hdst)

    pltpu.sync_copy(hsrc, hall.at[sid, 0])
    pltpu.sync_copy(hdst, hall.at[sid, 1])
    plsc.subcore_barrier()

    r0 = sid * _RN
    for k in range(16):
        pltpu.sync_copy(hall.at[k, 0, pl.ds(r0, _RN)], tmp)

        def addk(g, carry, _first=(k == 0)):
            sl = pl.ds(g * 16, 16)
            ds1[sl] = tmp[sl] if _first else ds1[sl] + tmp[sl]
            return carry

        lax.fori_loop(0, _RN // 16, addk, 0)
    for k in range(16):
        pltpu.sync_copy(hall.at[k, 1, pl.ds(r0, _RN)], tmp)

        def addk2(g, carry, _first=(k == 0)):
            sl = pl.ds(g * 16, 16)
            ds2[sl] = tmp[sl] if _first else ds2[sl] + tmp[sl]
            return carry

        lax.fori_loop(0, _RN // 16, addk2, 0)

    def nrm(g, carry):
        sl = pl.ds(g * 16, 16)
        y1 = _rsqrt16(jnp.maximum(ds1[sl], 1.0))
        y2 = _rsqrt16(jnp.maximum(ds2[sl], 1.0))
        ds1[sl] = y1
        ds2[sl] = y1 * y2
        tmp[sl] = y2
        return carry

    lax.fori_loop(0, _RN // 16, nrm, 0)
    pltpu.sync_copy(ds1, sn1.at[pl.ds(r0, _RN)])
    pltpu.sync_copy(ds2, sn2.at[pl.ds(r0, _RN)])

    @pl.when(cid == 0)
    def _():
        pltpu.sync_copy(tmp, innorm_hbm.at[pl.ds(r0, _RN)])

    plsc.subcore_barrier()
    pltpu.sync_copy(sn1, ntab1)
    pltpu.sync_copy(sn2, ntab2)

    base0 = wid * _NCHUNKS * _C

    def gathers(b):
        d1 = pltpu.async_copy(src_hbm.at[keptv[b]], srcv[b], sem)
        d2 = pltpu.async_copy(dst_hbm.at[keptv[b]], dstv[b], sem)
        d3 = pltpu.async_copy(w_hbm.at[keptv[b]], wv[b], sem)
        return d1, d2, d3

    def process(t, b):
        def coeff16(g, carry):
            sl = pl.ds(g * 16, 16)
            sidx = srcv[b][sl]
            w16 = wv[b][sl]
            cv[sl] = w16 * plsc.load_gather(ntab1, [sidx])
            c2v[sl] = w16 * plsc.load_gather(ntab2, [sidx])
            pkv[sl] = jnp.bitwise_or(jnp.left_shift(sidx, 16), dstv[b][sl])
            return carry

        lax.fori_loop(0, _C // 16, coeff16, 0)
        base = base0 + t * _C
        pltpu.sync_copy(pkv, fpk_hbm.at[pl.ds(base, _C)])
        pltpu.sync_copy(cv, fc1_hbm.at[pl.ds(base, _C)])
        pltpu.sync_copy(c2v, fc2_hbm.at[pl.ds(base, _C)])

    pltpu.sync_copy(kept_hbm.at[pl.ds(base0, _C)], keptv[0])
    paired = (_NCHUNKS // 2) * 2

    @pl.loop(0, paired, step=2)
    def pair(g):
        for b in range(2):
            t = g + b
            ds_ = gathers(b)

            @pl.when(t > 0)
            def _():
                process(t - 1, 1 - b)

            @pl.when(t + 1 < _NCHUNKS)
            def _():
                pltpu.sync_copy(kept_hbm.at[pl.ds(base0 + (t + 1) * _C, _C)],
                                keptv[1 - b])

            for d in ds_:
                d.wait()

    if _NCHUNKS % 2:
        ds_ = gathers(0)
        process(_NCHUNKS - 2, 1)
        for d in ds_:
            d.wait()

    process(_NCHUNKS - 1, (_NCHUNKS - 1) % 2)


def _layer_body(table_hbm, fpk_hbm, fc_hbm, out_hbm,
                srcv0, srcv1, dstv0, dstv1, cv0, cv1, pkv, rows0, rows1,
                zbuf, acc, sem, sem_s):
    cid = lax.axis_index("c")
    sid = lax.axis_index("s")
    wid = sid * 2 + cid

    srcv = (srcv0, srcv1)
    dstv = (dstv0, dstv1)
    cv = (cv0, cv1)
    rows = (rows0, rows1)

    row0 = sid * _RPT
    _zero_acc(zbuf, acc, row0)
    plsc.subcore_barrier()

    base0 = wid * _NCHUNKS * _C

    def load_idx(t, b):
        base = base0 + t * _C
        pltpu.sync_copy(fpk_hbm.at[pl.ds(base, _C)], pkv)
        pltpu.sync_copy(fc_hbm.at[pl.ds(base, _C)], cv[b])

        def unpack16(g, carry):
            sl = pl.ds(g * 16, 16)
            pk = pkv[sl]
            srcv[b][sl] = jnp.right_shift(pk, 16)
            dstv[b][sl] = jnp.bitwise_and(pk, 65535)
            return carry

        lax.fori_loop(0, _C // 16, unpack16, 0)

    def process(b):
        _scale_rows(rows[b], cv[b])
        pltpu.async_copy(rows[b], acc.at[dstv[b]], sem_s, add=True)

    def wait_scatter(b):
        pltpu.make_async_copy(rows[b], acc.at[dstv[b]], sem_s).wait()

    load_idx(0, 0)
    paired = (_NCHUNKS // 2) * 2

    @pl.loop(0, paired, step=2)
    def pair(g):
        for b in range(2):
            t = g + b

            @pl.when(t > 1)
            def _():
                wait_scatter(b)

            d = pltpu.async_copy(table_hbm.at[srcv[b]], rows[b], sem)

            @pl.when(t > 0)
            def _():
                process(1 - b)

            @pl.when(t + 1 < _NCHUNKS)
            def _():
                load_idx(t + 1, 1 - b)

            d.wait()

    if _NCHUNKS % 2:
        wait_scatter(0)
        d = pltpu.async_copy(table_hbm.at[srcv[0]], rows[0], sem)
        process(1)
        d.wait()

    last = (_NCHUNKS - 1) % 2
    wait_scatter(1 - last)
    _scale_rows(rows[last], cv[last])
    pltpu.sync_copy(rows[last], acc.at[dstv[last]], add=True)
    plsc.subcore_barrier()
    pltpu.sync_copy(acc.at[pl.ds(row0, _RPT)],
                    out_hbm.at[cid, pl.ds(row0, _RPT)])


_MESH = plsc.VectorSubcoreMesh(core_axis_name="c", subcore_axis_name="s")


@jax.jit
def _edge_filter(kept, srcx, dstx, wx):
    f = pl.kernel(
        _filter_body,
        out_type=(
            jax.ShapeDtypeStruct((_KP,), jnp.int32),
            jax.ShapeDtypeStruct((_KP,), jnp.float32),
            jax.ShapeDtypeStruct((_KP,), jnp.float32),
            jax.ShapeDtypeStruct((_NPAD,), jnp.float32),
        ),
        mesh=_MESH,
        scratch_types=[
            pltpu.VMEM((_C,), jnp.int32),
            pltpu.VMEM((_C,), jnp.int32),
            pltpu.VMEM((_C,), jnp.int32),
            pltpu.VMEM((_C,), jnp.int32),
            pltpu.VMEM((_C,), jnp.int32),
            pltpu.VMEM((_C,), jnp.int32),
            pltpu.VMEM((_C,), jnp.float32),
            pltpu.VMEM((_C,), jnp.float32),
            pltpu.VMEM((_C,), jnp.float32),
            pltpu.VMEM((_C,), jnp.float32),
            pltpu.VMEM((_C,), jnp.int32),
            pltpu.VMEM((_NPAD,), jnp.float32),
            pltpu.VMEM((_NPAD,), jnp.float32),
            pltpu.VMEM((_NPAD,), jnp.float32),
            pltpu.VMEM((_NPAD,), jnp.float32),
            pltpu.VMEM((_EB,), jnp.int32),
            pltpu.VMEM((_EB,), jnp.int32),
            pltpu.VMEM((_RN,), jnp.float32),
            pltpu.VMEM((_RN,), jnp.float32),
            pltpu.VMEM((_RN,), jnp.float32),
            pltpu.VMEM_SHARED((16, 2, _NPAD), jnp.float32),
            pltpu.VMEM_SHARED((_NPAD,), jnp.float32),
            pltpu.VMEM_SHARED((_NPAD,), jnp.float32),
            pltpu.SemaphoreType.DMA,
        ],
        compiler_params=pltpu.CompilerParams(needs_layout_passes=False),
    )
    return f(kept, srcx, dstx, wx)


@jax.jit
def _gcn_layer(table, fpk, fc):
    f = pl.kernel(
        _layer_body,
        out_type=jax.ShapeDtypeStruct((2, _NPAD, _D), jnp.float32),
        mesh=_MESH,
        scratch_types=[
            pltpu.VMEM((_C,), jnp.int32),
            pltpu.VMEM((_C,), jnp.int32),
            pltpu.VMEM((_C,), jnp.int32),
            pltpu.VMEM((_C,), jnp.int32),
            pltpu.VMEM((_C,), jnp.float32),
            pltpu.VMEM((_C,), jnp.float32),
            pltpu.VMEM((_C,), jnp.int32),
            pltpu.VMEM((_C, _D), jnp.float32),
            pltpu.VMEM((_C, _D), jnp.float32),
            pltpu.VMEM((_ZR, _D), jnp.float32),
            pltpu.VMEM_SHARED((_NPAD, _D), jnp.float32),
            pltpu.SemaphoreType.DMA,
            pltpu.SemaphoreType.DMA,
        ],
    )
    return f(table, fpk, fc)


def kernel(feature, edge_index, w):
    src = edge_index[0]
    dst = edge_index[1]

    pad_node = jnp.asarray(_PAD_NODE)
    srcx = jnp.concatenate([src, pad_node])
    dstx = jnp.concatenate([dst, pad_node])
    wx = jnp.concatenate([w, jnp.zeros((_PAD,), jnp.float32)])
    kept = jnp.asarray(_KEPT_PADDED)

    fpk, fc1, fc2, innorm = _edge_filter(kept, srcx, dstx, wx)
    s1p = _gcn_layer(feature, fpk, fc1)
    s1 = s1p[0, :_N] + s1p[1, :_N]
    s2p = _gcn_layer(s1, fpk, fc2)
    s2 = s2p[0, :_N] + s2p[1, :_N]

    final = (feature + innorm[:_N, None] * (s1 + s2)) * (1.0 / 3.0)
    return final


# speedup vs baseline: 1.0198x; 1.0198x over previous
import numpy as np
import jax
import jax.numpy as jnp
from jax import lax
from jax.experimental import pallas as pl
from jax.experimental.pallas import tpu as pltpu
from jax.experimental.pallas import tpu_sc as plsc

_N = 10000
_E = 320000
_D = 128
_DROP = int(0.7 * _E)

_C = 128
_NW = 32
_NPAD = 10240
_RPT = _NPAD // 16
_ZR = 80


def _threefry2x32(k0, k1, x0, x1):
    rotations = ((13, 15, 26, 6), (17, 29, 16, 24))
    k0 = np.uint32(k0); k1 = np.uint32(k1)
    ks = (k0, k1, k0 ^ k1 ^ np.uint32(0x1BD11BDA))
    x0 = (x0 + ks[0]).astype(np.uint32)
    x1 = (x1 + ks[1]).astype(np.uint32)
    for i in range(5):
        for r in rotations[i % 2]:
            x0 = (x0 + x1).astype(np.uint32)
            x1 = ((x1 << np.uint32(r)) | (x1 >> np.uint32(32 - r))).astype(np.uint32)
            x1 = x0 ^ x1
        x0 = (x0 + ks[(i + 1) % 3]).astype(np.uint32)
        x1 = (x1 + ks[(i + 2) % 3] + np.uint32(i + 1)).astype(np.uint32)
    return x0, x1


def _kept_indices() -> np.ndarray:
    def bits32(k, n):
        o0, o1 = _threefry2x32(k[0], k[1], np.zeros(n, np.uint32),
                               np.arange(n, dtype=np.uint32))
        return o0 ^ o1

    s0, s1 = _threefry2x32(np.uint32(0), np.uint32(1),
                           np.array([0, 0], np.uint32),
                           np.array([0, 1], np.uint32))
    k_hi = (s0[0], s1[0])
    k_lo = (s0[1], s1[1])
    span = np.uint32(_E)
    hi = bits32(k_hi, _DROP)
    lo = bits32(k_lo, _DROP)
    mult64 = (np.uint64(65536) % np.uint64(span)) ** 2 % np.uint64(2 ** 32)
    mult = np.uint32(mult64 % np.uint64(span))
    ridx = ((hi % span) * mult + (lo % span)).astype(np.uint32) % span
    mask = np.zeros((_E,), dtype=bool)
    mask[ridx] = True
    return np.nonzero(~mask)[0].astype(np.int32)


_KEPT = _kept_indices()
_K = int(_KEPT.shape[0])
_KP = ((_K + _NW * _C - 1) // (_NW * _C)) * (_NW * _C)
_PAD = _KP - _K
_PAD_NODE = ((np.arange(_PAD, dtype=np.int64) * 37) % _N).astype(np.int32)
_KEPT_PADDED = np.concatenate([_KEPT, _E + np.arange(_PAD, dtype=np.int32)])
_NCHUNKS = _KP // (_NW * _C)


def _zero_acc(zbuf, acc, row0):
    def zrow(i, carry):
        z = jnp.zeros((16,), jnp.float32)
        for j in range(8):
            zbuf[i, pl.ds(j * 16, 16)] = z
        return carry

    lax.fori_loop(0, _ZR, zrow, 0)
    for k in range(_RPT // _ZR):
        pltpu.sync_copy(zbuf, acc.at[pl.ds(row0 + k * _ZR, _ZR)])


def _scale_rows(rows, cv):
    def scale16(g, carry):
        cvec = cv[pl.ds(g * 16, 16)]
        for l in range(16):
            b = jnp.full((16,), cvec[l], jnp.float32)
            e = g * 16 + l
            for j in range(8):
                sl = pl.ds(j * 16, 16)
                rows[e, sl] = rows[e, sl] * b
        return carry

    lax.fori_loop(0, _C // 16, scale16, 0)


def _rsqrt16(x):
    i = plsc.bitcast(x, jnp.int32)
    i = 0x5F3759DF - jnp.right_shift(i, 1)
    y = plsc.bitcast(i, jnp.float32)
    for _ in range(3):
        y = y * (1.5 - 0.5 * x * y * y)
    return y


_EPT = _E // 16
_EB = 2000
_RN = _NPAD // 16


def _filter_body(kept_hbm, src_hbm, dst_hbm, w_hbm,
                 fpk_hbm, fc1_hbm, fc2_hbm, innorm_hbm,
                 keptv0, keptv1, srcv0, srcv1, dstv0, dstv1, wv0, wv1,
                 cv, c2v, pkv, ntab1, ntab2, hsrc, hdst, ebuf, ebuf2,
                 ds1, ds2, tmp, hall, sn1, sn2, sem):
    cid = lax.axis_index("c")
    sid = lax.axis_index("s")
    wid = sid * 2 + cid

    keptv = (keptv0, keptv1)
    srcv = (srcv0, srcv1)
    dstv = (dstv0, dstv1)
    wv = (wv0, wv1)

    def zrow(i, carry):
        z = jnp.zeros((16,), jnp.float32)
        hsrc[pl.ds(i * 16, 16)] = z
        hdst[pl.ds(i * 16, 16)] = z
        return carry

    lax.fori_loop(0, _NPAD // 16, zrow, 0)

    ones = jnp.ones((16,), jnp.float32)

    def hist16(buf, hist):
        @pl.loop(0, _EB // 16, unroll=4)
        def h16(g):
            idx = buf[pl.ds(g * 16, 16)]
            plsc.addupdate_scatter(hist, [idx], ones)

    pltpu.sync_copy(src_hbm.at[pl.ds(sid * _EPT, _EB)], ebuf)

    @pl.loop(0, _EPT // _EB)
    def hist_chunk(c):
        base = sid * _EPT + c * _EB
        d = pltpu.async_copy(dst_hbm.at[pl.ds(base, _EB)], ebuf2, sem)
        hist16(ebuf, hsrc)
        d.wait()

        @pl.when(c + 1 < _EPT // _EB)
        def _():
            d2 = pltpu.async_copy(
                src_hbm.at[pl.ds(base + _EB, _EB)], ebuf, sem)
            hist16(ebuf2, hdst)
            d2.wait()

        @pl.when(c + 1 >= _EPT // _EB)
        def _():
            hist16(ebuf2, hdst)

    pltpu.sync_copy(hsrc, hall.at[sid, 0])
    pltpu.sync_copy(hdst, hall.at[sid, 1])
    plsc.subcore_barrier()

    r0 = sid * _RN
    for k in range(16):
        pltpu.sync_copy(hall.at[k, 0, pl.ds(r0, _RN)], tmp)

        def addk(g, carry, _first=(k == 0)):
            sl = pl.ds(g * 16, 16)
            ds1[sl] = tmp[sl] if _first else ds1[sl] + tmp[sl]
            return carry

        lax.fori_loop(0, _RN // 16, addk, 0)
    for k in range(16):
        pltpu.sync_copy(hall.at[k, 1, pl.ds(r0, _RN)], tmp)

        def addk2(g, carry, _first=(k == 0)):
            sl = pl.ds(g * 16, 16)
            ds2[sl] = tmp[sl] if _first else ds2[sl] + tmp[sl]
            return carry

        lax.fori_loop(0, _RN // 16, addk2, 0)

    def nrm(g, carry):
        sl = pl.ds(g * 16, 16)
        y1 = _rsqrt16(jnp.maximum(ds1[sl], 1.0))
        y2 = _rsqrt16(jnp.maximum(ds2[sl], 1.0))
        ds1[sl] = y1
        ds2[sl] = y1 * y2
        tmp[sl] = y2
        return carry

    lax.fori_loop(0, _RN // 16, nrm, 0)
    pltpu.sync_copy(ds1, sn1.at[pl.ds(r0, _RN)])
    pltpu.sync_copy(ds2, sn2.at[pl.ds(r0, _RN)])

    @pl.when(cid == 0)
    def _():
        pltpu.sync_copy(tmp, innorm_hbm.at[pl.ds(r0, _RN)])

    plsc.subcore_barrier()
    pltpu.sync_copy(sn1, ntab1)
    pltpu.sync_copy(sn2, ntab2)

    base0 = wid * _NCHUNKS * _C

    def gathers(b):
        d1 = pltpu.async_copy(src_hbm.at[keptv[b]], srcv[b], sem)
        d2 = pltpu.async_copy(dst_hbm.at[keptv[b]], dstv[b], sem)
        d3 = pltpu.async_copy(w_hbm.at[keptv[b]], wv[b], sem)
        return d1, d2, d3

    def process(t, b):
        def coeff16(g, carry):
            sl = pl.ds(g * 16, 16)
            sidx = srcv[b][sl]
            w16 = wv[b][sl]
            cv[sl] = w16 * plsc.load_gather(ntab1, [sidx])
            c2v[sl] = w16 * plsc.load_gather(ntab2, [sidx])
            pkv[sl] = jnp.bitwise_or(jnp.left_shift(sidx, 16), dstv[b][sl])
            return carry

        lax.fori_loop(0, _C // 16, coeff16, 0)
        base = base0 + t * _C
        pltpu.sync_copy(pkv, fpk_hbm.at[pl.ds(base, _C)])
        pltpu.sync_copy(cv, fc1_hbm.at[pl.ds(base, _C)])
        pltpu.sync_copy(c2v, fc2_hbm.at[pl.ds(base, _C)])

    pltpu.sync_copy(kept_hbm.at[pl.ds(base0, _C)], keptv[0])
    paired = (_NCHUNKS // 2) * 2

    @pl.loop(0, paired, step=2)
    def pair(g):
        for b in range(2):
            t = g + b
            ds_ = gathers(b)

            @pl.when(t > 0)
            def _():
                process(t - 1, 1 - b)

            @pl.when(t + 1 < _NCHUNKS)
            def _():
                pltpu.sync_copy(kept_hbm.at[pl.ds(base0 + (t + 1) * _C, _C)],
                                keptv[1 - b])

            for d in ds_:
                d.wait()

    if _NCHUNKS % 2:
        ds_ = gathers(0)
        process(_NCHUNKS - 2, 1)
        for d in ds_:
            d.wait()

    process(_NCHUNKS - 1, (_NCHUNKS - 1) % 2)


def _layer_body(table_hbm, fpk_hbm, fc_hbm, out_hbm,
                srcv0, srcv1, dstv0, dstv1, cv0, cv1, pkv, rows0, rows1,
                zbuf, acc, sem, sem_s):
    cid = lax.axis_index("c")
    sid = lax.axis_index("s")
    wid = sid * 2 + cid

    srcv = (srcv0, srcv1)
    dstv = (dstv0, dstv1)
    cv = (cv0, cv1)
    rows = (rows0, rows1)

    row0 = sid * _RPT
    _zero_acc(zbuf, acc, row0)
    plsc.subcore_barrier()

    base0 = wid * _NCHUNKS * _C

    def load_idx(t, b):
        base = base0 + t * _C
        pltpu.sync_copy(fpk_hbm.at[pl.ds(base, _C)], pkv)
        pltpu.sync_copy(fc_hbm.at[pl.ds(base, _C)], cv[b])

        def unpack16(g, carry):
            sl = pl.ds(g * 16, 16)
            pk = pkv[sl]
            srcv[b][sl] = jnp.right_shift(pk, 16)
            dstv[b][sl] = jnp.bitwise_and(pk, 65535)
            return carry

        lax.fori_loop(0, _C // 16, unpack16, 0)

    def process(b):
        _scale_rows(rows[b], cv[b])
        pltpu.async_copy(rows[b], acc.at[dstv[b]], sem_s, add=True)

    def wait_scatter(b):
        pltpu.make_async_copy(rows[b], acc.at[dstv[b]], sem_s).wait()

    load_idx(0, 0)
    paired = (_NCHUNKS // 2) * 2

    @pl.loop(0, paired, step=2)
    def pair(g):
        for b in range(2):
            t = g + b

            @pl.when(t > 1)
            def _():
                wait_scatter(b)

            d = pltpu.async_copy(table_hbm.at[srcv[b]], rows[b], sem)

            @pl.when(t > 0)
            def _():
                process(1 - b)

            @pl.when(t + 1 < _NCHUNKS)
            def _():
                load_idx(t + 1, 1 - b)

            d.wait()

    if _NCHUNKS % 2:
        wait_scatter(0)
        d = pltpu.async_copy(table_hbm.at[srcv[0]], rows[0], sem)
        process(1)
        d.wait()

    last = (_NCHUNKS - 1) % 2
    wait_scatter(1 - last)
    _scale_rows(rows[last], cv[last])
    pltpu.sync_copy(rows[last], acc.at[dstv[last]], add=True)
    plsc.subcore_barrier()
    pltpu.sync_copy(acc.at[pl.ds(row0, _RPT)],
                    out_hbm.at[cid, pl.ds(row0, _RPT)])


_MESH = plsc.VectorSubcoreMesh(core_axis_name="c", subcore_axis_name="s")


@jax.jit
def _edge_filter(kept, srcx, dstx, wx):
    f = pl.kernel(
        _filter_body,
        out_type=(
            jax.ShapeDtypeStruct((_KP,), jnp.int32),
            jax.ShapeDtypeStruct((_KP,), jnp.float32),
            jax.ShapeDtypeStruct((_KP,), jnp.float32),
            jax.ShapeDtypeStruct((_NPAD,), jnp.float32),
        ),
        mesh=_MESH,
        scratch_types=[
            pltpu.VMEM((_C,), jnp.int32),
            pltpu.VMEM((_C,), jnp.int32),
            pltpu.VMEM((_C,), jnp.int32),
            pltpu.VMEM((_C,), jnp.int32),
            pltpu.VMEM((_C,), jnp.int32),
            pltpu.VMEM((_C,), jnp.int32),
            pltpu.VMEM((_C,), jnp.float32),
            pltpu.VMEM((_C,), jnp.float32),
            pltpu.VMEM((_C,), jnp.float32),
            pltpu.VMEM((_C,), jnp.float32),
            pltpu.VMEM((_C,), jnp.int32),
            pltpu.VMEM((_NPAD,), jnp.float32),
            pltpu.VMEM((_NPAD,), jnp.float32),
            pltpu.VMEM((_NPAD,), jnp.float32),
            pltpu.VMEM((_NPAD,), jnp.float32),
            pltpu.VMEM((_EB,), jnp.int32),
            pltpu.VMEM((_EB,), jnp.int32),
            pltpu.VMEM((_RN,), jnp.float32),
            pltpu.VMEM((_RN,), jnp.float32),
            pltpu.VMEM((_RN,), jnp.float32),
            pltpu.VMEM_SHARED((16, 2, _NPAD), jnp.float32),
            pltpu.VMEM_SHARED((_NPAD,), jnp.float32),
            pltpu.VMEM_SHARED((_NPAD,), jnp.float32),
            pltpu.SemaphoreType.DMA,
        ],
        compiler_params=pltpu.CompilerParams(needs_layout_passes=False),
    )
    return f(kept, srcx, dstx, wx)


@jax.jit
def _gcn_layer(table, fpk, fc):
    f = pl.kernel(
        _layer_body,
        out_type=jax.ShapeDtypeStruct((2, _NPAD, _D), jnp.float32),
        mesh=_MESH,
        scratch_types=[
            pltpu.VMEM((_C,), jnp.int32),
            pltpu.VMEM((_C,), jnp.int32),
            pltpu.VMEM((_C,), jnp.int32),
            pltpu.VMEM((_C,), jnp.int32),
            pltpu.VMEM((_C,), jnp.float32),
            pltpu.VMEM((_C,), jnp.float32),
            pltpu.VMEM((_C,), jnp.int32),
            pltpu.VMEM((_C, _D), jnp.float32),
            pltpu.VMEM((_C, _D), jnp.float32),
            pltpu.VMEM((_ZR, _D), jnp.float32),
            pltpu.VMEM_SHARED((_NPAD, _D), jnp.float32),
            pltpu.SemaphoreType.DMA,
            pltpu.SemaphoreType.DMA,
        ],
    )
    return f(table, fpk, fc)


def kernel(feature, edge_index, w):
    src = edge_index[0]
    dst = edge_index[1]

    pad_node = jnp.asarray(_PAD_NODE)
    srcx = jnp.concatenate([src, pad_node])
    dstx = jnp.concatenate([dst, pad_node])
    wx = jnp.concatenate([w, jnp.zeros((_PAD,), jnp.float32)])
    kept = jnp.asarray(_KEPT_PADDED)

    fpk, fc1, fc2, innorm = _edge_filter(kept, srcx, dstx, wx)
    s1p = _gcn_layer(feature, fpk, fc1)
    s1 = s1p[0, :_N] + s1p[1, :_N]
    s2p = _gcn_layer(s1, fpk, fc2)
    s2 = s2p[0, :_N] + s2p[1, :_N]

    final = (feature + innorm[:_N, None] * (s1 + s2)) * (1.0 / 3.0)
    return final
